# Initial kernel scaffold; baseline (speedup 1.0000x reference)
import jax
import jax.numpy as jnp
from jax.experimental import pallas as pl


def kernel(x, W0, W1, W2):
    raise NotImplementedError("write your pallas kernel here")


# speedup vs baseline: 2.0600x; 2.0600x over previous
import functools

import jax
import jax.numpy as jnp
from jax import lax
from jax.experimental import pallas as pl
from jax.experimental.pallas import tpu as pltpu
from jax.experimental.pallas import tpu_sc as plsc

_B = 1024
_E = 200
_EMB = 64
_NCH = 3
_N = _B * _E * _NCH
_NW = 32
_PER_W = _N // _NW
_K = 128
_CHUNKS = _PER_W // _K
_OFF1 = 129
_OFF2 = 258


@functools.partial(
    pl.kernel,
    mesh=plsc.VectorSubcoreMesh(core_axis_name="c", subcore_axis_name="s"),
    out_type=jax.ShapeDtypeStruct((_N, _EMB), jnp.float32),
    compiler_params=pltpu.CompilerParams(use_tc_tiling_on_sc=False),
    scratch_types=[
        pltpu.VMEM((_K,), jnp.int32),
        pltpu.VMEM((_K,), jnp.int32),
        pltpu.VMEM((_K, _EMB), jnp.float32),
        pltpu.SemaphoreType.DMA,
    ],
)
def _embed(idx_hbm, w_hbm, out_hbm, idx_raw, idx_adj, rows, sem):
    wid = lax.axis_index("s") * 2 + lax.axis_index("c")
    base = wid * _PER_W
    lane = lax.iota(jnp.int32, 16)

    def chunk_body(ch, _):
        off = base + ch * _K
        pltpu.sync_copy(idx_hbm.at[pl.ds(off, _K)], idx_raw)

        def vreg_body(j, _):
            pos = off + j * 16 + lane
            r = lax.rem(pos, 3)
            adj = idx_raw[pl.ds(j * 16, 16)] + jnp.where(
                r == 1, _OFF1, jnp.where(r == 2, _OFF2, 0)
            )
            idx_adj[pl.ds(j * 16, 16)] = adj
            return 0

        lax.fori_loop(0, _K // 16, vreg_body, 0)
        pltpu.async_copy(w_hbm.at[idx_adj], rows, sem).wait()
        pltpu.sync_copy(rows, out_hbm.at[pl.ds(off, _K)])
        return 0

    lax.fori_loop(0, _CHUNKS, chunk_body, 0)


def kernel(x, W0, W1, W2):
    w = jnp.concatenate([W0, W1, W2], axis=0)
    idx = x.reshape(_N)
    out = _embed(idx, w)
    return out.reshape(_B, _E, _NCH, _EMB)


# speedup vs baseline: 2.0865x; 1.0128x over previous
import functools

import jax
import jax.numpy as jnp
from jax import lax
from jax.experimental import pallas as pl
from jax.experimental.pallas import tpu as pltpu
from jax.experimental.pallas import tpu_sc as plsc

_B = 1024
_E = 200
_EMB = 64
_NCH = 3
_N = _B * _E * _NCH
_NW = 32
_PER_W = _N // _NW
_K = 128
_CH = 640
_NSTR = _CH // _K
_CHUNKS = _PER_W // _CH
_VPC = _CH // 16


@functools.partial(
    pl.kernel,
    mesh=plsc.VectorSubcoreMesh(core_axis_name="c", subcore_axis_name="s"),
    out_type=jax.ShapeDtypeStruct((_N, _EMB), jnp.float32),
    compiler_params=pltpu.CompilerParams(use_tc_tiling_on_sc=False),
    scratch_types=[
        pltpu.VMEM((_CH,), jnp.int32),
        pltpu.VMEM((2, _CH), jnp.int32),
        pltpu.VMEM((2, _CH, _EMB), jnp.float32),
        pltpu.SemaphoreType.DMA,
        pltpu.SemaphoreType.DMA,
        pltpu.SemaphoreType.DMA,
    ],
)
def _embed(idx_hbm, w_hbm, out_hbm, idx_raw, idx_adj, rows, sem_g, sem_o0, sem_o1):
    wid = lax.axis_index("s") * 2 + lax.axis_index("c")
    base = wid * _PER_W
    lane = lax.iota(jnp.int32, 16)
    sem_o = (sem_o0, sem_o1)

    def stage_idx(c, b):
        off = base + c * _CH
        pltpu.sync_copy(idx_hbm.at[pl.ds(off, _CH)], idx_raw)

        def vreg_body(j, _):
            pos = off + j * 16 + lane
            adj = idx_raw[pl.ds(j * 16, 16)] + 129 * lax.rem(pos, 3)
            idx_adj[b, pl.ds(j * 16, 16)] = adj
            return 0

        lax.fori_loop(0, _VPC, vreg_body, 0)

    def fire_gathers(b):
        for k in range(_NSTR):
            pltpu.async_copy(
                w_hbm.at[idx_adj.at[b, pl.ds(k * _K, _K)]],
                rows.at[b, pl.ds(k * _K, _K)],
                sem_g,
            )

    def wait_gathers(b):
        pltpu.make_async_copy(w_hbm.at[idx_adj.at[b]], rows.at[b], sem_g).wait()

    def fire_out(c, b):
        pltpu.async_copy(rows.at[b], out_hbm.at[pl.ds(base + c * _CH, _CH)], sem_o[b])

    def wait_out(c, b):
        pltpu.make_async_copy(
            rows.at[b], out_hbm.at[pl.ds(base + c * _CH, _CH)], sem_o[b]
        ).wait()

    def pair_body(i, _):
        for b in range(2):
            c = 2 * i + b
            @pl.when(i >= 1)
            def _():
                wait_out(c - 2, b)

            stage_idx(c, b)
            if b == 0:
                @pl.when(i >= 1)
                def _():
                    wait_gathers(1)
            else:
                wait_gathers(0)
            fire_gathers(b)
            if b == 0:
                @pl.when(i >= 1)
                def _():
                    fire_out(c - 1, 1)
            else:
                fire_out(c - 1, 0)
        return 0

    lax.fori_loop(0, _CHUNKS // 2, pair_body, 0)

    wait_gathers(1)
    fire_out(_CHUNKS - 1, 1)
    wait_out(_CHUNKS - 2, 0)
    wait_out(_CHUNKS - 1, 1)


def kernel(x, W0, W1, W2):
    w = jnp.concatenate([W0, W1, W2], axis=0)
    idx = x.reshape(_N)
    out = _embed(idx, w)
    return out.reshape(_B, _E, _NCH, _EMB)


# speedup vs baseline: 2.3245x; 1.1141x over previous
import functools

import jax
import jax.numpy as jnp
from jax import lax
from jax.experimental import pallas as pl
from jax.experimental.pallas import tpu as pltpu
from jax.experimental.pallas import tpu_sc as plsc

_B = 1024
_E = 200
_EMB = 64
_NCH = 3
_NW = 32
_NUNITS = _E * _NCH
_U_LO = _NUNITS // _NW
_U_EXTRA = _NUNITS % _NW
_WFLAT = (129 + 129 + 106) * _EMB


@functools.partial(
    pl.kernel,
    mesh=plsc.VectorSubcoreMesh(core_axis_name="c", subcore_axis_name="s"),
    out_type=jax.ShapeDtypeStruct((_E, _NCH, 8, 8, 8, 128), jnp.float32),
    compiler_params=pltpu.CompilerParams(
        use_tc_tiling_on_sc=False, needs_layout_passes=False
    ),
    scratch_types=[
        pltpu.VMEM((_WFLAT,), jnp.float32),
        pltpu.VMEM((8, 128), jnp.int32),
        pltpu.VMEM((2, 8, 4, 8, 128), jnp.float32),
        pltpu.SemaphoreType.DMA,
        pltpu.SemaphoreType.DMA,
    ],
)
def _embed(x5, w1d, out6, wv, idxv, buf, sem0, sem1):
    wid = lax.axis_index("s") * 2 + lax.axis_index("c")
    pltpu.sync_copy(w1d, wv)

    n_u = _U_LO + (wid < _U_EXTRA).astype(jnp.int32)
    start = _U_LO * wid + jnp.minimum(wid, _U_EXTRA)
    sems = (sem0, sem1)

    def unit_body(u, _):
        p = start + u
        c = p // _E
        e = p - c * _E
        e8 = e // 8
        e0 = e - e8 * 8
        pltpu.sync_copy(x5.at[c, e8, :, e0, :], idxv)
        coff = c * 129 * _EMB

        for h in range(2):
            @pl.when(u >= 1)
            def _():
                pltpu.make_async_copy(
                    buf.at[h],
                    out6.at[e, c, :, pl.ds(h * 4, 4), :, :],
                    sems[h],
                ).wait()

            for b7h in range(4):
                b7 = h * 4 + b7h

                def g_body(g, _, b7=b7, b7h=b7h, h=h):
                    iv = idxv[b7, pl.ds(g * 16, 16)]
                    base = iv * _EMB + coff
                    for j in range(_EMB):
                        v = plsc.load_gather(wv, [base + j])
                        buf[h, j // 8, b7h, j % 8, pl.ds(g * 16, 16)] = v
                    return 0

                lax.fori_loop(0, 8, g_body, 0)

            pltpu.async_copy(
                buf.at[h], out6.at[e, c, :, pl.ds(h * 4, 4), :, :], sems[h]
            )
        return 0

    lax.fori_loop(0, n_u, unit_body, 0)
    for h in range(2):
        pltpu.make_async_copy(
            buf.at[h], out6.at[0, 0, :, pl.ds(h * 4, 4), :, :], sems[h]
        ).wait()


def kernel(x, W0, W1, W2):
    w = jnp.concatenate([W0, W1, W2], axis=0).reshape(_WFLAT)
    x5 = jnp.transpose(x.reshape(8, 128, 25, 8, 3), (4, 2, 0, 3, 1))
    z = _embed(x5, w)
    out = jnp.transpose(z, (3, 5, 0, 1, 2, 4))
    return out.reshape(_B, _E, _NCH, _EMB)


# speedup vs baseline: 16.7878x; 7.2220x over previous
import functools

import jax
import jax.numpy as jnp
from jax import lax
from jax.experimental import pallas as pl
from jax.experimental.pallas import tpu as pltpu
from jax.experimental.pallas import tpu_sc as plsc

_B = 1024
_E = 200
_EMB = 64
_NCH = 3
_NW = 32
_NUNITS = _E * _NCH
_U_LO = _NUNITS // _NW
_U_EXTRA = _NUNITS % _NW
_VOCAB = 129 + 129 + 106
_WSTRIDE = _EMB + 1
_WFLAT = _VOCAB * _WSTRIDE


@functools.partial(
    pl.kernel,
    mesh=plsc.VectorSubcoreMesh(core_axis_name="c", subcore_axis_name="s"),
    out_type=jax.ShapeDtypeStruct((_E, _NCH, 8, 8, 8, 128), jnp.float32),
    compiler_params=pltpu.CompilerParams(
        use_tc_tiling_on_sc=False, needs_layout_passes=False
    ),
    scratch_types=[
        pltpu.VMEM((_WFLAT,), jnp.float32),
        pltpu.VMEM((8, 128), jnp.int32),
        pltpu.VMEM((2, 8, 4, 8, 128), jnp.float32),
        pltpu.SemaphoreType.DMA,
        pltpu.SemaphoreType.DMA,
    ],
)
def _embed(x5, w1d, out6, wv, idxv, buf, sem0, sem1):
    wid = lax.axis_index("s") * 2 + lax.axis_index("c")
    pltpu.sync_copy(w1d, wv)

    n_u = _U_LO + (wid < _U_EXTRA).astype(jnp.int32)
    start = _U_LO * wid + jnp.minimum(wid, _U_EXTRA)
    sems = (sem0, sem1)

    def unit_body(u, _):
        p = start + u
        c = p // _E
        e = p - c * _E
        e8 = e // 8
        e0 = e - e8 * 8
        pltpu.sync_copy(x5.at[c, e8, :, e0, :], idxv)
        coff = c * 129 * _WSTRIDE

        for h in range(2):
            @pl.when(u >= 1)
            def _():
                pltpu.make_async_copy(
                    buf.at[h],
                    out6.at[e, c, :, pl.ds(h * 4, 4), :, :],
                    sems[h],
                ).wait()

            for b7h in range(4):
                b7 = h * 4 + b7h

                def g_body(g, _, b7=b7, b7h=b7h, h=h):
                    iv = idxv[b7, pl.ds(g * 16, 16)]
                    base = iv * _WSTRIDE + coff
                    for j0 in range(0, _EMB, 8):
                        vs = [
                            plsc.load_gather(wv, [base + (j0 + t)])
                            for t in range(8)
                        ]
                        for t in range(8):
                            j = j0 + t
                            buf[h, j // 8, b7h, j % 8, pl.ds(g * 16, 16)] = vs[t]
                    return 0

                lax.fori_loop(0, 8, g_body, 0)

            pltpu.async_copy(
                buf.at[h], out6.at[e, c, :, pl.ds(h * 4, 4), :, :], sems[h]
            )
        return 0

    lax.fori_loop(0, n_u, unit_body, 0)
    for h in range(2):
        pltpu.make_async_copy(
            buf.at[h], out6.at[0, 0, :, pl.ds(h * 4, 4), :, :], sems[h]
        ).wait()


def kernel(x, W0, W1, W2):
    w = jnp.pad(
        jnp.concatenate([W0, W1, W2], axis=0), ((0, 0), (0, 1))
    ).reshape(_WFLAT)
    x5 = jnp.transpose(x.reshape(8, 128, 25, 8, 3), (4, 2, 0, 3, 1))
    z = _embed(x5, w)
    out = jnp.transpose(z, (3, 5, 0, 1, 2, 4))
    return out.reshape(_B, _E, _NCH, _EMB)


# speedup vs baseline: 18.6493x; 1.1109x over previous
import functools

import jax
import jax.numpy as jnp
from jax import lax
from jax.experimental import pallas as pl
from jax.experimental.pallas import tpu as pltpu
from jax.experimental.pallas import tpu_sc as plsc

_B = 1024
_E = 200
_EMB = 64
_NCH = 3
_NW = 32
_NUNITS = _E * _NCH
_U_LO = _NUNITS // _NW
_U_EXTRA = _NUNITS % _NW
_VOCAB = 129 + 129 + 106
_WSTRIDE = _EMB + 1
_WFLAT = _VOCAB * _WSTRIDE


@functools.partial(
    pl.kernel,
    mesh=plsc.VectorSubcoreMesh(core_axis_name="c", subcore_axis_name="s"),
    out_type=jax.ShapeDtypeStruct((_E, _NCH, 8, 8, 8, 128), jnp.float32),
    compiler_params=pltpu.CompilerParams(
        use_tc_tiling_on_sc=False, needs_layout_passes=False
    ),
    scratch_types=[
        pltpu.VMEM((_WFLAT,), jnp.float32),
        pltpu.VMEM((4, 8, 8, 128), jnp.int32),
        pltpu.VMEM((2, 8, 4, 8, 128), jnp.float32),
        pltpu.SemaphoreType.DMA,
        pltpu.SemaphoreType.DMA,
        pltpu.SemaphoreType.DMA,
    ],
)
def _embed(x4, w1d, out6, wv, idxa, buf, sem0, sem1, semi):
    wid = lax.axis_index("s") * 2 + lax.axis_index("c")

    n_u = _U_LO + (wid < _U_EXTRA).astype(jnp.int32)
    start = _U_LO * wid + jnp.minimum(wid, _U_EXTRA)
    sems = (sem0, sem1)

    f = start // 8
    idx_handles = [
        pltpu.async_copy(x4.at[jnp.minimum(f + k, 74)], idxa.at[k], semi)
        for k in range(4)
    ]
    pltpu.sync_copy(w1d, wv)
    for h in idx_handles:
        h.wait()

    def unit_body(u, _):
        p = start + u
        c = p // _E
        e = p - c * _E
        e8 = e // 8
        e0 = e - e8 * 8
        bl = (c * 25 + e8) - f
        coff = c * 129 * _WSTRIDE

        for h in range(2):
            @pl.when(u >= 1)
            def _():
                pltpu.make_async_copy(
                    buf.at[h],
                    out6.at[e, c, :, pl.ds(h * 4, 4), :, :],
                    sems[h],
                ).wait()

            for b7h in range(4):
                b7 = h * 4 + b7h

                def g_body(g, _, b7=b7, b7h=b7h, h=h):
                    iv = idxa[bl, b7, e0, pl.ds(g * 16, 16)]
                    base = iv * _WSTRIDE + coff
                    for j0 in range(0, _EMB, 8):
                        vs = [
                            plsc.load_gather(wv, [base + (j0 + t)])
                            for t in range(8)
                        ]
                        for t in range(8):
                            j = j0 + t
                            buf[h, j // 8, b7h, j % 8, pl.ds(g * 16, 16)] = vs[t]
                    return 0

                lax.fori_loop(0, 8, g_body, 0)

            pltpu.async_copy(
                buf.at[h], out6.at[e, c, :, pl.ds(h * 4, 4), :, :], sems[h]
            )
        return 0

    lax.fori_loop(0, n_u, unit_body, 0)
    for h in range(2):
        pltpu.make_async_copy(
            buf.at[h], out6.at[0, 0, :, pl.ds(h * 4, 4), :, :], sems[h]
        ).wait()


def kernel(x, W0, W1, W2):
    w = jnp.pad(
        jnp.concatenate([W0, W1, W2], axis=0), ((0, 0), (0, 1))
    ).reshape(_WFLAT)
    x4 = jnp.transpose(x.reshape(8, 128, 25, 8, 3), (4, 2, 0, 3, 1)).reshape(
        75, 8, 8, 128
    )
    z = _embed(x4, w)
    out = jnp.transpose(z, (3, 5, 0, 1, 2, 4))
    return out.reshape(_B, _E, _NCH, _EMB)
